# Initial kernel scaffold; baseline (speedup 1.0000x reference)
import jax
import jax.numpy as jnp
from jax.experimental import pallas as pl


def kernel(x, edge_attr, edge_index, W_node, b_node, W_edge, b_edge, Wa1, ba1, Wa2, ba2):
    raise NotImplementedError("write your pallas kernel here")


# speedup vs baseline: 5.1392x; 5.1392x over previous
import functools

import jax
import jax.numpy as jnp
from jax import lax
from jax.experimental import pallas as pl
from jax.experimental.pallas import tpu as pltpu
from jax.experimental.pallas import tpu_sc as plsc

N = 10000
E = 640000
CH = 128

NC = 2
NS = 16
NW = NC * NS
EPW = E // NW
K = 80
NCHUNK = EPW // K


def _node_tables(x, W_node, b_node, W1t, W1b, ba1):
    BN = 1000
    grid = (N // BN,)

    def body(x_ref, wn_ref, bn_ref, w1t_ref, w1b_ref, ba1_ref, a_ref, b_ref):
        h = jnp.dot(x_ref[...], wn_ref[...], preferred_element_type=jnp.float32)
        h = h + bn_ref[...]
        a_ref[...] = jnp.dot(h, w1t_ref[...], preferred_element_type=jnp.float32) + ba1_ref[...]
        b_ref[...] = jnp.dot(h, w1b_ref[...], preferred_element_type=jnp.float32)

    return pl.pallas_call(
        body,
        grid=grid,
        in_specs=[
            pl.BlockSpec((BN, x.shape[1]), lambda i: (i, 0)),
            pl.BlockSpec((x.shape[1], CH), lambda i: (0, 0)),
            pl.BlockSpec((1, CH), lambda i: (0, 0)),
            pl.BlockSpec((CH, CH), lambda i: (0, 0)),
            pl.BlockSpec((CH, CH), lambda i: (0, 0)),
            pl.BlockSpec((1, CH), lambda i: (0, 0)),
        ],
        out_specs=[
            pl.BlockSpec((BN, CH), lambda i: (i, 0)),
            pl.BlockSpec((BN, CH), lambda i: (i, 0)),
        ],
        out_shape=[
            jax.ShapeDtypeStruct((N, CH), jnp.float32),
            jax.ShapeDtypeStruct((N, CH), jnp.float32),
        ],
    )(x, W_node, b_node.reshape(1, CH), W1t, W1b, ba1.reshape(1, CH))


def _edge_scores(a_tab, b_tab, src, dst, w2, ba2v):
    mesh = plsc.VectorSubcoreMesh(core_axis_name="c", subcore_axis_name="s")

    @functools.partial(
        pl.kernel,
        mesh=mesh,
        out_type=jax.ShapeDtypeStruct((E,), jnp.float32),
        compiler_params=pltpu.CompilerParams(needs_layout_passes=False),
        scratch_types=[
            pltpu.VMEM((K,), jnp.int32),
            pltpu.VMEM((K,), jnp.int32),
            pltpu.VMEM((K, CH), jnp.float32),
            pltpu.VMEM((K, CH), jnp.float32),
            pltpu.VMEM((K,), jnp.float32),
            pltpu.VMEM((K * 16,), jnp.float32),
            pltpu.VMEM((CH,), jnp.float32),
            pltpu.VMEM((16,), jnp.float32),
            pltpu.SemaphoreType.DMA,
            pltpu.SemaphoreType.DMA,
        ],
    )
    def k(a_hbm, b_hbm, src_hbm, dst_hbm, w2_hbm, ba2_hbm, out_hbm,
          idx_s, idx_d, rows_a, rows_b, out_v, accbuf, w2_v, ba2_v, sem_a, sem_b):
        wid = lax.axis_index("s") * NC + lax.axis_index("c")
        base = wid * EPW
        pltpu.sync_copy(w2_hbm, w2_v)
        pltpu.sync_copy(ba2_hbm, ba2_v)

        def chunk_body(j, _):
            off = base + j * K
            pltpu.sync_copy(src_hbm.at[pl.ds(off, K)], idx_s)
            pltpu.sync_copy(dst_hbm.at[pl.ds(off, K)], idx_d)
            cp_a = pltpu.async_copy(a_hbm.at[idx_s], rows_a, sem_a)
            cp_b = pltpu.async_copy(b_hbm.at[idx_d], rows_b, sem_b)
            cp_a.wait()
            cp_b.wait()

            def edge_body(e, _):
                acc = ba2_v[...]
                for cg in range(CH // 16):
                    va = rows_a[e, pl.ds(cg * 16, 16)]
                    vb = rows_b[e, pl.ds(cg * 16, 16)]
                    s = va + vb
                    l = jnp.maximum(s, s * jnp.float32(0.01))
                    acc = acc + l * w2_v[pl.ds(cg * 16, 16)]
                accbuf[pl.ds(e * 16, 16)] = acc
                return 0

            lax.fori_loop(0, K, edge_body, 0)

            lane16 = lax.iota(jnp.int32, 16) * 16
            for g in range(K // 16):
                base_idx = lane16 + g * 256
                t = jnp.zeros((16,), jnp.float32)
                for c in range(16):
                    t = t + plsc.load_gather(accbuf, [base_idx + c])
                out_v[pl.ds(g * 16, 16)] = t
            pltpu.sync_copy(out_v, out_hbm.at[pl.ds(off, K)])
            return 0

        lax.fori_loop(0, NCHUNK, chunk_body, 0)

    return k(a_tab, b_tab, src, dst, w2, ba2v)


def kernel(x, edge_attr, edge_index, W_node, b_node, W_edge, b_edge,
           Wa1, ba1, Wa2, ba2):
    del edge_attr, W_edge, b_edge
    src = edge_index[0]
    dst = edge_index[1]
    W1t = Wa1[:CH]
    W1b = Wa1[CH:]
    a_tab, b_tab = _node_tables(x, W_node, b_node, W1t, W1b, ba1)
    w2 = Wa2.reshape(CH)
    ba2v = jnp.zeros((16,), jnp.float32).at[0].set(ba2[0])
    out = _edge_scores(a_tab, b_tab, src, dst, w2, ba2v)
    return out.reshape(E, 1)


# speedup vs baseline: 14.9520x; 2.9094x over previous
import functools

import jax
import jax.numpy as jnp
from jax import lax
from jax.experimental import pallas as pl
from jax.experimental.pallas import tpu as pltpu
from jax.experimental.pallas import tpu_sc as plsc

N = 10000
E = 640000
CH = 128

NC = 2
NS = 16
NW = NC * NS
EPW = E // NW
K = 80
NCHUNK = EPW // K


def _node_tables(x, W_node, b_node, W1t, W1b, ba1):
    BN = 1000
    grid = (N // BN,)

    def body(x_ref, wn_ref, bn_ref, w1t_ref, w1b_ref, ba1_ref, a_ref, b_ref):
        h = jnp.dot(x_ref[...], wn_ref[...], preferred_element_type=jnp.float32)
        h = h + bn_ref[...]
        a_ref[...] = jnp.dot(h, w1t_ref[...], preferred_element_type=jnp.float32) + ba1_ref[...]
        b_ref[...] = jnp.dot(h, w1b_ref[...], preferred_element_type=jnp.float32)

    return pl.pallas_call(
        body,
        grid=grid,
        in_specs=[
            pl.BlockSpec((BN, x.shape[1]), lambda i: (i, 0)),
            pl.BlockSpec((x.shape[1], CH), lambda i: (0, 0)),
            pl.BlockSpec((1, CH), lambda i: (0, 0)),
            pl.BlockSpec((CH, CH), lambda i: (0, 0)),
            pl.BlockSpec((CH, CH), lambda i: (0, 0)),
            pl.BlockSpec((1, CH), lambda i: (0, 0)),
        ],
        out_specs=[
            pl.BlockSpec((BN, CH), lambda i: (i, 0)),
            pl.BlockSpec((BN, CH), lambda i: (i, 0)),
        ],
        out_shape=[
            jax.ShapeDtypeStruct((N, CH), jnp.float32),
            jax.ShapeDtypeStruct((N, CH), jnp.float32),
        ],
    )(x, W_node, b_node.reshape(1, CH), W1t, W1b, ba1.reshape(1, CH))


def _edge_scores(a_tab, b_tab, src, dst, w2, ba2v):
    mesh = plsc.VectorSubcoreMesh(core_axis_name="c", subcore_axis_name="s")

    @functools.partial(
        pl.kernel,
        mesh=mesh,
        out_type=jax.ShapeDtypeStruct((E,), jnp.float32),
        compiler_params=pltpu.CompilerParams(needs_layout_passes=False),
        scratch_types=[
            pltpu.VMEM((EPW,), jnp.int32),
            pltpu.VMEM((EPW,), jnp.int32),
            pltpu.VMEM((2, K, CH), jnp.float32),
            pltpu.VMEM((2, K, CH), jnp.float32),
            pltpu.VMEM((EPW,), jnp.float32),
            pltpu.VMEM((K * 16,), jnp.float32),
            pltpu.VMEM((CH,), jnp.float32),
            pltpu.VMEM((16,), jnp.float32),
            pltpu.SemaphoreType.DMA,
            pltpu.SemaphoreType.DMA,
            pltpu.SemaphoreType.DMA,
            pltpu.SemaphoreType.DMA,
        ],
    )
    def k(a_hbm, b_hbm, src_hbm, dst_hbm, w2_hbm, ba2_hbm, out_hbm,
          idx_s, idx_d, rows_a, rows_b, out_all, accbuf, w2_v, ba2_v,
          sa0, sa1, sb0, sb1):
        wid = lax.axis_index("s") * NC + lax.axis_index("c")
        base = wid * EPW
        sem_a = [sa0, sa1]
        sem_b = [sb0, sb1]
        pltpu.sync_copy(w2_hbm, w2_v)
        pltpu.sync_copy(ba2_hbm, ba2_v)
        pltpu.sync_copy(src_hbm.at[pl.ds(base, EPW)], idx_s)
        pltpu.sync_copy(dst_hbm.at[pl.ds(base, EPW)], idx_d)

        def gather_issue(j, b):
            pltpu.async_copy(a_hbm.at[idx_s.at[pl.ds(j * K, K)]],
                             rows_a.at[b], sem_a[b])
            pltpu.async_copy(b_hbm.at[idx_d.at[pl.ds(j * K, K)]],
                             rows_b.at[b], sem_b[b])

        def gather_wait(j, b):
            pltpu.make_async_copy(a_hbm.at[idx_s.at[pl.ds(j * K, K)]],
                                  rows_a.at[b], sem_a[b]).wait()
            pltpu.make_async_copy(b_hbm.at[idx_d.at[pl.ds(j * K, K)]],
                                  rows_b.at[b], sem_b[b]).wait()

        lane16 = lax.iota(jnp.int32, 16) * 16

        def compute_chunk(j, b):
            def edge_body(e):
                acc = ba2_v[...]
                for cg in range(CH // 16):
                    va = rows_a[b, e, pl.ds(cg * 16, 16)]
                    vb = rows_b[b, e, pl.ds(cg * 16, 16)]
                    s = va + vb
                    l = jnp.maximum(s, s * jnp.float32(0.01))
                    acc = acc + l * w2_v[pl.ds(cg * 16, 16)]
                accbuf[pl.ds(e * 16, 16)] = acc

            plsc.parallel_loop(0, K, unroll=4)(edge_body)

            for g in range(K // 16):
                base_idx = lane16 + g * 256
                t = jnp.zeros((16,), jnp.float32)
                for c in range(16):
                    t = t + plsc.load_gather(accbuf, [base_idx + c])
                out_all[pl.ds(j * K + g * 16, 16)] = t

        gather_issue(0, 0)

        @pl.loop(0, NCHUNK // 2)
        def pair_body(i):
            j0 = i * 2
            gather_issue(j0 + 1, 1)
            gather_wait(j0, 0)
            compute_chunk(j0, 0)
            j2 = jnp.where(j0 + 2 < NCHUNK, j0 + 2, 0)
            gather_issue(j2, 0)
            gather_wait(j0 + 1, 1)
            compute_chunk(j0 + 1, 1)

        gather_wait(0, 0)
        pltpu.sync_copy(out_all, out_hbm.at[pl.ds(base, EPW)])

    return k(a_tab, b_tab, src, dst, w2, ba2v)


def kernel(x, edge_attr, edge_index, W_node, b_node, W_edge, b_edge,
           Wa1, ba1, Wa2, ba2):
    del edge_attr, W_edge, b_edge
    src = edge_index[0]
    dst = edge_index[1]
    W1t = Wa1[:CH]
    W1b = Wa1[CH:]
    a_tab, b_tab = _node_tables(x, W_node, b_node, W1t, W1b, ba1)
    w2 = Wa2.reshape(CH)
    ba2v = jnp.zeros((16,), jnp.float32).at[0].set(ba2[0])
    out = _edge_scores(a_tab, b_tab, src, dst, w2, ba2v)
    return out.reshape(E, 1)
